# baseline (speedup 1.0000x reference)
import functools

import jax
import jax.numpy as jnp
from jax import lax
from jax.experimental import pallas as pl
from jax.experimental.pallas import tpu as pltpu
from jax.experimental.pallas import tpu_sc as plsc

N_NODES = 10000
H = 128
O = 128
N_RELS = 8
N_BASES = 4
N_EDGES = 320000

NC = 2
NS = 16
NW = NC * NS
E_PER_W = N_EDGES // NW
CHUNK = 40
N_CHUNKS = E_PER_W // CHUNK
ACC_ROWS = 10240
ROWS_PER_S = ACC_ROWS // NS


def _make_y(x, V, comp):
    nb = 10
    bm = N_NODES // nb

    def body(comp_ref, v_ref, x_ref, y_ref):
        xb = x_ref[...]
        for r in range(N_RELS):
            w = jnp.sum(comp_ref[r][:, None, None] * v_ref[...], axis=0)
            y_ref[r] = jnp.dot(xb, w, preferred_element_type=jnp.float32)

    return pl.pallas_call(
        body,
        grid=(nb,),
        in_specs=[
            pl.BlockSpec((N_RELS, N_BASES), lambda n: (0, 0)),
            pl.BlockSpec((N_BASES, H, O), lambda n: (0, 0, 0)),
            pl.BlockSpec((bm, H), lambda n: (n, 0)),
        ],
        out_specs=pl.BlockSpec((N_RELS, bm, O), lambda n: (0, n, 0)),
        out_shape=jax.ShapeDtypeStruct((N_RELS, N_NODES, O), jnp.float32),
    )(comp, V, x)


def _sc_aggregate(y, src_r, dst_r, et_r):
    mesh = plsc.VectorSubcoreMesh(core_axis_name="c", subcore_axis_name="s",
                                  num_cores=NC, num_subcores=NS)

    @functools.partial(
        pl.kernel,
        mesh=mesh,
        compiler_params=pltpu.CompilerParams(use_tc_tiling_on_sc=False),
        out_type=(
            jax.ShapeDtypeStruct((NC, ACC_ROWS, O), jnp.float32),
            jax.ShapeDtypeStruct((NC, ACC_ROWS, 16), jnp.float32),
        ),
        scratch_types=[
            pltpu.VMEM((2000,), jnp.int32),
            pltpu.VMEM((2000,), jnp.int32),
            pltpu.VMEM((E_PER_W,), jnp.int32),
            pltpu.VMEM((N_CHUNKS, CHUNK), jnp.int32),
            pltpu.VMEM((CHUNK, 16), jnp.float32),
            pltpu.VMEM((CHUNK, 16), jnp.float32),
            pltpu.VMEM((2, CHUNK, O), jnp.float32),
            pltpu.VMEM_SHARED((ACC_ROWS, O), jnp.float32),
            pltpu.VMEM_SHARED((ACC_ROWS, 16), jnp.float32),
            pltpu.SemaphoreType.DMA,
        ],
    )
    def body(y_hbm, src_hbm, dst_hbm, et_hbm, acc_out, deg_out,
             src_b, et_b, gidx_b, dst_b, z16, ones_b, rows_b,
             acc_sh, deg_sh, sem):
        c = lax.axis_index("c")
        s = lax.axis_index("s")
        wid = c * NS + s
        base = s * ROWS_PER_S

        zeros16 = jnp.zeros((16,), jnp.float32)
        ones16 = jnp.ones((16,), jnp.float32)

        def z_rows(i, _):
            rows_b[0, i // 8, pl.ds((i % 8) * 16, 16)] = zeros16
            return 0
        lax.fori_loop(0, CHUNK * (O // 16), z_rows, 0)

        def z_deg(i, _):
            z16[i] = zeros16
            ones_b[i] = ones16
            return 0
        lax.fori_loop(0, CHUNK, z_deg, 0)

        for k in range(ROWS_PER_S // CHUNK):
            pltpu.sync_copy(rows_b.at[0], acc_sh.at[pl.ds(base + k * CHUNK, CHUNK)])
            pltpu.sync_copy(z16, deg_sh.at[pl.ds(base + k * CHUNK, CHUNK)])

        pltpu.sync_copy(dst_hbm.at[wid], dst_b)
        for blk in range(E_PER_W // 2000):
            pltpu.sync_copy(src_hbm.at[wid, pl.ds(blk * 2000, 2000)], src_b)
            pltpu.sync_copy(et_hbm.at[wid, pl.ds(blk * 2000, 2000)], et_b)

            def gidx(i, _, blk=blk):
                sl = pl.ds(i * 16, 16)
                osl = pl.ds(blk * 2000 + i * 16, 16)
                gidx_b[osl] = et_b[sl] * N_NODES + src_b[sl]
                return 0
            lax.fori_loop(0, 125, gidx, 0)

        plsc.subcore_barrier()

        pltpu.async_copy(y_hbm.at[gidx_b.at[pl.ds(0, CHUNK)]], rows_b.at[0],
                         sem)

        def outer(t, _):
            for b in range(2):
                j = 2 * t + b
                pltpu.make_async_copy(
                    y_hbm.at[gidx_b.at[pl.ds(0, CHUNK)]], rows_b.at[b],
                    sem).wait()

                @pl.when(j < N_CHUNKS - 1)
                def _():
                    idx = gidx_b.at[pl.ds((j + 1) * CHUNK, CHUNK)]
                    pltpu.async_copy(y_hbm.at[idx], rows_b.at[1 - b], sem)

                pltpu.sync_copy(rows_b.at[b], acc_sh.at[dst_b.at[j]],
                                add=True)
                pltpu.sync_copy(ones_b, deg_sh.at[dst_b.at[j]], add=True)
            return 0
        lax.fori_loop(0, N_CHUNKS // 2, outer, 0)

        plsc.subcore_barrier()

        pltpu.sync_copy(acc_sh.at[pl.ds(base, ROWS_PER_S)],
                        acc_out.at[c, pl.ds(base, ROWS_PER_S)])
        pltpu.sync_copy(deg_sh.at[pl.ds(base, ROWS_PER_S)],
                        deg_out.at[c, pl.ds(base, ROWS_PER_S)])

    return body(y, src_r, dst_r, et_r)


def _finalize(acc, deg):
    nb = 5
    bm = N_NODES // nb

    def body(a_ref, d_ref, o_ref):
        d = d_ref[0, :, 0:1] + d_ref[1, :, 0:1]
        norm = 1.0 / jnp.maximum(d, 1.0)
        o_ref[...] = jnp.maximum((a_ref[0] + a_ref[1]) * norm, 0.0)

    return pl.pallas_call(
        body,
        grid=(nb,),
        in_specs=[
            pl.BlockSpec((NC, bm, O), lambda n: (0, n, 0)),
            pl.BlockSpec((NC, bm, 16), lambda n: (0, n, 0)),
        ],
        out_specs=pl.BlockSpec((bm, O), lambda n: (n, 0)),
        out_shape=jax.ShapeDtypeStruct((N_NODES, O), jnp.float32),
    )(acc, deg)


def kernel(x, edge_index, edge_type, V, comp):
    src_r = edge_index[0].astype(jnp.int32).reshape(NW, E_PER_W)
    dst_r = edge_index[1].astype(jnp.int32).reshape(NW, N_CHUNKS, CHUNK)
    et_r = edge_type.astype(jnp.int32).reshape(NW, E_PER_W)
    y = _make_y(x, V, comp).reshape(N_RELS * N_NODES, O)
    acc, deg = _sc_aggregate(y, src_r, dst_r, et_r)
    return _finalize(acc, deg)


# speedup vs baseline: 1.0219x; 1.0219x over previous
import functools

import jax
import jax.numpy as jnp
from jax import lax
from jax.experimental import pallas as pl
from jax.experimental.pallas import tpu as pltpu
from jax.experimental.pallas import tpu_sc as plsc

N_NODES = 10000
H = 128
O = 128
N_RELS = 8
N_BASES = 4
N_EDGES = 320000

NC = 2
NS = 16
NW = NC * NS
E_PER_W = N_EDGES // NW
CHUNK = 80
N_CHUNKS = E_PER_W // CHUNK
ACC_ROWS = 10240
ROWS_PER_S = ACC_ROWS // NS


def _make_y(x, V, comp):
    nb = 10
    bm = N_NODES // nb

    def body(comp_ref, v_ref, x_ref, y_ref):
        xb = x_ref[...]
        for r in range(N_RELS):
            w = jnp.sum(comp_ref[r][:, None, None] * v_ref[...], axis=0)
            y_ref[r] = jnp.dot(xb, w, preferred_element_type=jnp.float32)

    return pl.pallas_call(
        body,
        grid=(nb,),
        in_specs=[
            pl.BlockSpec((N_RELS, N_BASES), lambda n: (0, 0)),
            pl.BlockSpec((N_BASES, H, O), lambda n: (0, 0, 0)),
            pl.BlockSpec((bm, H), lambda n: (n, 0)),
        ],
        out_specs=pl.BlockSpec((N_RELS, bm, O), lambda n: (0, n, 0)),
        out_shape=jax.ShapeDtypeStruct((N_RELS, N_NODES, O), jnp.float32),
    )(comp, V, x)


def _sc_aggregate(y, src_r, dst_r, et_r):
    mesh = plsc.VectorSubcoreMesh(core_axis_name="c", subcore_axis_name="s",
                                  num_cores=NC, num_subcores=NS)

    @functools.partial(
        pl.kernel,
        mesh=mesh,
        compiler_params=pltpu.CompilerParams(use_tc_tiling_on_sc=False, needs_layout_passes=False),
        out_type=(
            jax.ShapeDtypeStruct((NC, ACC_ROWS, O), jnp.float32),
            jax.ShapeDtypeStruct((NW, ROWS_PER_S, 16), jnp.float32),
        ),
        scratch_types=[
            pltpu.VMEM((E_PER_W,), jnp.int32),
            pltpu.VMEM((2000,), jnp.int32),
            pltpu.VMEM((N_CHUNKS, CHUNK), jnp.int32),
            pltpu.VMEM((CHUNK, O), jnp.float32),
            pltpu.VMEM((ROWS_PER_S, 16), jnp.float32),
            pltpu.VMEM_SHARED((ACC_ROWS, O), jnp.float32),
            pltpu.SemaphoreType.DMA,
        ],
    )
    def body(y_hbm, src_hbm, dst_hbm, et_hbm, acc_out, deg_out,
             gidx_b, et_b, dst_b, rows_b, hist, acc_sh, sem):
        c = lax.axis_index("c")
        s = lax.axis_index("s")
        wid = c * NS + s
        base = s * ROWS_PER_S

        zeros16 = jnp.zeros((16,), jnp.float32)
        ones16 = jnp.ones((16,), jnp.float32)

        def z_rows(i, _):
            rows_b[i // 8, pl.ds((i % 8) * 16, 16)] = zeros16
            return 0
        lax.fori_loop(0, CHUNK * (O // 16), z_rows, 0)

        def z_hist(i, _):
            hist[i] = zeros16
            return 0
        lax.fori_loop(0, ROWS_PER_S, z_hist, 0)

        for k in range(ROWS_PER_S // CHUNK):
            pltpu.sync_copy(rows_b, acc_sh.at[pl.ds(base + k * CHUNK, CHUNK)])

        pltpu.sync_copy(src_hbm.at[wid], gidx_b)
        pltpu.sync_copy(dst_hbm.at[wid], dst_b)
        for blk in range(E_PER_W // 2000):
            pltpu.sync_copy(et_hbm.at[wid, pl.ds(blk * 2000, 2000)], et_b)

            def gidx(i, _, blk=blk):
                sl = pl.ds(blk * 2000 + i * 16, 16)
                gidx_b[sl] = et_b[pl.ds(i * 16, 16)] * N_NODES + gidx_b[sl]
                return 0
            lax.fori_loop(0, 125, gidx, 0)

        def hist_up(i, _):
            d = dst_b[i // (CHUNK // 16), pl.ds((i % (CHUNK // 16)) * 16, 16)]
            plsc.addupdate_scatter(hist, [d >> 4, d & 15], ones16)
            return 0
        lax.fori_loop(0, E_PER_W // 16, hist_up, 0)

        plsc.subcore_barrier()

        def chunk(j, _):
            idx = gidx_b.at[pl.ds(j * CHUNK, CHUNK)]
            pltpu.async_copy(y_hbm.at[idx], rows_b, sem).wait()
            pltpu.sync_copy(rows_b, acc_sh.at[dst_b.at[j]], add=True)
            return 0
        lax.fori_loop(0, N_CHUNKS, chunk, 0)

        plsc.subcore_barrier()

        pltpu.sync_copy(acc_sh.at[pl.ds(base, ROWS_PER_S)],
                        acc_out.at[c, pl.ds(base, ROWS_PER_S)])
        pltpu.sync_copy(hist, deg_out.at[wid])

    return body(y, src_r, dst_r, et_r)


def _finalize(acc, deg):
    nb = 5
    bm = ACC_ROWS // nb

    def body(a_ref, d_ref, o_ref):
        d = jnp.sum(d_ref[...], axis=0)[:, None]
        norm = 1.0 / jnp.maximum(d, 1.0)
        o_ref[...] = jnp.maximum((a_ref[0] + a_ref[1]) * norm, 0.0)

    return pl.pallas_call(
        body,
        grid=(nb,),
        in_specs=[
            pl.BlockSpec((NC, bm, O), lambda n: (0, n, 0)),
            pl.BlockSpec((NW, bm), lambda n: (0, n)),
        ],
        out_specs=pl.BlockSpec((bm, O), lambda n: (n, 0)),
        out_shape=jax.ShapeDtypeStruct((ACC_ROWS, O), jnp.float32),
    )(acc, deg)


def kernel(x, edge_index, edge_type, V, comp):
    src_r = edge_index[0].astype(jnp.int32).reshape(NW, E_PER_W)
    dst_r = edge_index[1].astype(jnp.int32).reshape(NW, N_CHUNKS, CHUNK)
    et_r = edge_type.astype(jnp.int32).reshape(NW, E_PER_W)
    y = _make_y(x, V, comp).reshape(N_RELS * N_NODES, O)
    acc, deg = _sc_aggregate(y, src_r, dst_r, et_r)
    return _finalize(acc, deg.reshape(NW, ACC_ROWS))[:N_NODES]


# speedup vs baseline: 1.2558x; 1.2289x over previous
import functools

import jax
import jax.numpy as jnp
from jax import lax
from jax.experimental import pallas as pl
from jax.experimental.pallas import tpu as pltpu
from jax.experimental.pallas import tpu_sc as plsc

N_NODES = 10000
H = 128
O = 128
N_RELS = 8
N_BASES = 4
N_EDGES = 320000

NC = 2
NS = 16
NW = NC * NS
E_PER_W = N_EDGES // NW
CHUNK = 80
N_CHUNKS = E_PER_W // CHUNK
ACC_ROWS = 10240
ROWS_PER_S = ACC_ROWS // NS


def _make_y(x, V, comp):
    nb = 10
    bm = N_NODES // nb

    def body(comp_ref, v_ref, x_ref, y_ref):
        xb = x_ref[...]
        for r in range(N_RELS):
            w = jnp.sum(comp_ref[r][:, None, None] * v_ref[...], axis=0)
            y_ref[r] = jnp.dot(xb, w, preferred_element_type=jnp.float32)

    return pl.pallas_call(
        body,
        grid=(nb,),
        in_specs=[
            pl.BlockSpec((N_RELS, N_BASES), lambda n: (0, 0)),
            pl.BlockSpec((N_BASES, H, O), lambda n: (0, 0, 0)),
            pl.BlockSpec((bm, H), lambda n: (n, 0)),
        ],
        out_specs=pl.BlockSpec((N_RELS, bm, O), lambda n: (0, n, 0)),
        out_shape=jax.ShapeDtypeStruct((N_RELS, N_NODES, O), jnp.float32),
    )(comp, V, x)


def _sc_aggregate(y, src_r, dst_r, et_r):
    mesh = plsc.VectorSubcoreMesh(core_axis_name="c", subcore_axis_name="s",
                                  num_cores=NC, num_subcores=NS)

    @functools.partial(
        pl.kernel,
        mesh=mesh,
        compiler_params=pltpu.CompilerParams(use_tc_tiling_on_sc=False, needs_layout_passes=False),
        out_type=(
            jax.ShapeDtypeStruct((NC, ACC_ROWS, O), jnp.float32),
            jax.ShapeDtypeStruct((NW, ROWS_PER_S, 16), jnp.float32),
        ),
        scratch_types=[
            pltpu.VMEM((E_PER_W,), jnp.int32),
            pltpu.VMEM((2000,), jnp.int32),
            pltpu.VMEM((2, CHUNK), jnp.int32),
            pltpu.VMEM((2, CHUNK, O), jnp.float32),
            pltpu.VMEM((ROWS_PER_S, 16), jnp.float32),
            pltpu.VMEM_SHARED((ACC_ROWS, O), jnp.float32),
            pltpu.SemaphoreType.DMA,
            pltpu.SemaphoreType.DMA,
        ],
    )
    def body(y_hbm, src_hbm, dst_hbm, et_hbm, acc_out, deg_out,
             gidx_b, et_b, dst_b, rows_b, hist, acc_sh, gsem, dsem):
        c = lax.axis_index("c")
        s = lax.axis_index("s")
        wid = c * NS + s
        base = s * ROWS_PER_S

        zeros16 = jnp.zeros((16,), jnp.float32)
        ones16 = jnp.ones((16,), jnp.float32)

        def z_rows(i, _):
            rows_b[0, i // 8, pl.ds((i % 8) * 16, 16)] = zeros16
            return 0
        lax.fori_loop(0, CHUNK * (O // 16), z_rows, 0)

        def z_hist(i, _):
            hist[i] = zeros16
            return 0
        lax.fori_loop(0, ROWS_PER_S, z_hist, 0)

        for k in range(ROWS_PER_S // CHUNK):
            pltpu.sync_copy(rows_b.at[0], acc_sh.at[pl.ds(base + k * CHUNK, CHUNK)])

        pltpu.sync_copy(src_hbm.at[wid], gidx_b)
        for blk in range(E_PER_W // 2000):
            pltpu.sync_copy(et_hbm.at[wid, pl.ds(blk * 2000, 2000)], et_b)

            def gidx(i, _, blk=blk):
                sl = pl.ds(blk * 2000 + i * 16, 16)
                gidx_b[sl] = et_b[pl.ds(i * 16, 16)] * N_NODES + gidx_b[sl]
                return 0
            lax.fori_loop(0, 125, gidx, 0)

        plsc.subcore_barrier()

        pltpu.async_copy(dst_hbm.at[wid, 0], dst_b.at[0], dsem)
        pltpu.async_copy(y_hbm.at[gidx_b.at[pl.ds(0, CHUNK)]], rows_b.at[0],
                         gsem)

        def outer(t, _):
            for b in range(2):
                j = 2 * t + b
                o = 1 - b
                pltpu.make_async_copy(dst_hbm.at[wid, 0], dst_b.at[b],
                                      dsem).wait()
                pltpu.make_async_copy(
                    y_hbm.at[gidx_b.at[pl.ds(0, CHUNK)]], rows_b.at[b],
                    gsem).wait()

                @pl.when(j < N_CHUNKS - 1)
                def _():
                    pltpu.async_copy(dst_hbm.at[wid, j + 1], dst_b.at[o],
                                     dsem)
                    idx = gidx_b.at[pl.ds((j + 1) * CHUNK, CHUNK)]
                    pltpu.async_copy(y_hbm.at[idx], rows_b.at[o], gsem)

                pltpu.sync_copy(rows_b.at[b], acc_sh.at[dst_b.at[b]],
                                add=True)

                def hist_up(i, _):
                    d = dst_b[b, pl.ds(i * 16, 16)]
                    plsc.addupdate_scatter(hist, [d >> 4, d & 15], ones16)
                    return 0
                lax.fori_loop(0, CHUNK // 16, hist_up, 0)
            return 0
        lax.fori_loop(0, N_CHUNKS // 2, outer, 0)

        last = N_CHUNKS - 1
        lb = last % 2
        pltpu.make_async_copy(dst_hbm.at[wid, 0], dst_b.at[lb], dsem).wait()
        pltpu.make_async_copy(
            y_hbm.at[gidx_b.at[pl.ds(0, CHUNK)]], rows_b.at[lb], gsem).wait()
        pltpu.sync_copy(rows_b.at[lb], acc_sh.at[dst_b.at[lb]], add=True)

        def hist_tail(i, _):
            d = dst_b[lb, pl.ds(i * 16, 16)]
            plsc.addupdate_scatter(hist, [d >> 4, d & 15], ones16)
            return 0
        lax.fori_loop(0, CHUNK // 16, hist_tail, 0)

        plsc.subcore_barrier()

        pltpu.sync_copy(acc_sh.at[pl.ds(base, ROWS_PER_S)],
                        acc_out.at[c, pl.ds(base, ROWS_PER_S)])
        pltpu.sync_copy(hist, deg_out.at[wid])

    return body(y, src_r, dst_r, et_r)


def _finalize(acc, deg):
    nb = 5
    bm = ACC_ROWS // nb

    def body(a_ref, d_ref, o_ref):
        d = jnp.sum(d_ref[...], axis=0)[:, None]
        norm = 1.0 / jnp.maximum(d, 1.0)
        o_ref[...] = jnp.maximum((a_ref[0] + a_ref[1]) * norm, 0.0)

    return pl.pallas_call(
        body,
        grid=(nb,),
        in_specs=[
            pl.BlockSpec((NC, bm, O), lambda n: (0, n, 0)),
            pl.BlockSpec((NW, bm), lambda n: (0, n)),
        ],
        out_specs=pl.BlockSpec((bm, O), lambda n: (n, 0)),
        out_shape=jax.ShapeDtypeStruct((ACC_ROWS, O), jnp.float32),
    )(acc, deg)


def kernel(x, edge_index, edge_type, V, comp):
    src_r = edge_index[0].astype(jnp.int32).reshape(NW, E_PER_W)
    dst_r = edge_index[1].astype(jnp.int32).reshape(NW, N_CHUNKS, CHUNK)
    et_r = edge_type.astype(jnp.int32).reshape(NW, E_PER_W)
    y = _make_y(x, V, comp).reshape(N_RELS * N_NODES, O)
    acc, deg = _sc_aggregate(y, src_r, dst_r, et_r)
    return _finalize(acc, deg.reshape(NW, ACC_ROWS))[:N_NODES]
